# baseline (speedup 1.0000x reference)
import functools

import jax
import jax.numpy as jnp
from jax import lax
from jax.experimental import pallas as pl
from jax.experimental.pallas import tpu as pltpu
from jax.experimental.pallas import tpu_sc as plsc

_F = 26
_D = 16
_B = 16384
_V1 = 100001
_NC = 2
_NS = 16
_L = 16
_NW = _NC * _NS
_BPW = _B // _NW
_CH = 128
_NCH = _BPW // _CH

_TBLK = 4096
_TGRID = -(-_V1 // _TBLK)
_ROWS_PAD = _TGRID * _TBLK
_HPB = _TBLK // 1024


def _make_relayout(nt):
    def body(*refs):
        ins, outs = refs[:nt], refs[nt:]
        for t in range(nt):
            x = ins[t][...]
            for h in range(_HPB):
                c = jnp.concatenate(
                    [x[:, (8 * h + j) * 128:(8 * h + j + 1) * 128]
                     for j in range(8)], axis=0)
                outs[t][h * 128:(h + 1) * 128, :] = c.T

    return pl.pallas_call(
        body,
        grid=(_TGRID,),
        in_specs=[pl.BlockSpec((_D, _TBLK), lambda j: (0, j))] * nt,
        out_specs=[pl.BlockSpec((_HPB * 128, 128), lambda j: (j, 0))] * nt,
        out_shape=[
            jax.ShapeDtypeStruct((_TGRID * _HPB * 128, 128), jnp.float32)
        ] * nt,
    )


_PF = 32


def _post_body(x_ref, o_ref):
    c = x_ref[...].T
    cr = c.reshape(16, 8, 128)
    for fr in range(8):
        for g in range(2):
            o_ref[fr, g, 0, :, :] = cr[2 * fr + g]


_postlayout = pl.pallas_call(
    _post_body,
    grid=(_B // 128, 4),
    in_specs=[pl.BlockSpec((128, 128), lambda bt, j: (4 * bt + j, 0))],
    out_specs=pl.BlockSpec((8, 2, 1, 8, 128), lambda bt, j: (j, 0, bt, 0, 0)),
    out_shape=jax.ShapeDtypeStruct((_F, 2, _B // 128, 8, 128), jnp.float32),
)


def _make_emb_kernel(nf):
    def body(*refs):
        feats = refs[:nf]
        tables = refs[nf:2 * nf]
        out = refs[2 * nf]
        idx_v, rows_v, sem_idx, sem_g, sem_w = refs[2 * nf + 1:]

        wid = lax.axis_index("s") * _NC + lax.axis_index("c")
        base = wid * _BPW

        idx_copies = [
            pltpu.make_async_copy(
                feats[i].at[pl.ds(base + c * _CH, _CH)],
                idx_v.at[i, c], sem_idx)
            for i in range(nf)
            for c in range(_NCH)
        ]
        for cp in idx_copies:
            cp.start()
        for cp in idx_copies:
            cp.wait()

        for i in range(nf):
            def rbody(it, _, i=i):
                c = it >> 3
                k = it & 7
                v = idx_v[i, c, pl.ds(k * _L, _L)]
                idx_v[i, c, pl.ds(k * _L, _L)] = (
                    (v & -1024) + ((v & 127) << 3) + ((v >> 7) & 7))
                return 0

            lax.fori_loop(0, _NCH * (_CH // _L), rbody, 0)

        def gather_descs(i, p):
            return [
                pltpu.make_async_copy(
                    tables[i].at[idx_v.at[i, c]],
                    rows_v.at[p, pl.ds(c * _CH, _CH)],
                    sem_g.at[p])
                for c in range(_NCH)
            ]

        def write_desc(i, p):
            return pltpu.make_async_copy(
                rows_v.at[p],
                out.at[pl.ds(base, _BPW), pl.ds(i * _D, _D)],
                sem_w.at[p])

        for i in range(nf):
            p = i % 2
            if i >= 2:
                write_desc(i - 2, p).wait()
            for g in gather_descs(i, p):
                g.start()
            if i >= 1:
                q = 1 - p
                for g in gather_descs(i - 1, q):
                    g.wait()
                write_desc(i - 1, q).start()

        p_last = (nf - 1) % 2
        for g in gather_descs(nf - 1, p_last):
            g.wait()
        write_desc(nf - 1, p_last).start()
        write_desc(nf - 2, 1 - p_last).wait()
        write_desc(nf - 1, p_last).wait()

    return functools.partial(
        pl.kernel,
        mesh=plsc.VectorSubcoreMesh(core_axis_name="c", subcore_axis_name="s"),
        out_type=jax.ShapeDtypeStruct((_B, _PF * _D), jnp.float32),
        scratch_types=[
            pltpu.VMEM((nf, _NCH, _CH), jnp.int32),
            pltpu.VMEM((2, _BPW, _D), jnp.float32),
            pltpu.SemaphoreType.DMA,
            pltpu.SemaphoreType.DMA((2,)),
            pltpu.SemaphoreType.DMA((2,)),
        ],
        compiler_params=pltpu.CompilerParams(use_tc_tiling_on_sc=False),
    )(lambda *refs: body(*refs))


_relayout_full = _make_relayout(_F)
_emb_full = _make_emb_kernel(_F)


def kernel(feat_0, feat_1, feat_2, feat_3, feat_4, feat_5, feat_6, feat_7,
           feat_8, feat_9, feat_10, feat_11, feat_12, feat_13, feat_14,
           feat_15, feat_16, feat_17, feat_18, feat_19, feat_20, feat_21,
           feat_22, feat_23, feat_24, feat_25,
           W_0, W_1, W_2, W_3, W_4, W_5, W_6, W_7, W_8, W_9, W_10, W_11,
           W_12, W_13, W_14, W_15, W_16, W_17, W_18, W_19, W_20, W_21,
           W_22, W_23, W_24, W_25):
    feats = [
        feat_0, feat_1, feat_2, feat_3, feat_4, feat_5, feat_6, feat_7,
        feat_8, feat_9, feat_10, feat_11, feat_12, feat_13, feat_14,
        feat_15, feat_16, feat_17, feat_18, feat_19, feat_20, feat_21,
        feat_22, feat_23, feat_24, feat_25,
    ]
    tables = [
        W_0, W_1, W_2, W_3, W_4, W_5, W_6, W_7, W_8, W_9, W_10, W_11,
        W_12, W_13, W_14, W_15, W_16, W_17, W_18, W_19, W_20, W_21,
        W_22, W_23, W_24, W_25,
    ]
    d = _relayout_full(*[w.T for w in tables])
    dense = [t.reshape(_ROWS_PAD, _D) for t in d]
    out = _emb_full(*feats, *dense)
    out5 = _postlayout(out.reshape(_B * 4, 128))
    return out5.transpose(2, 4, 0, 1, 3).reshape(_B, _F, _D)


# speedup vs baseline: 1.0003x; 1.0003x over previous
import functools

import jax
import jax.numpy as jnp
from jax import lax
from jax.experimental import pallas as pl
from jax.experimental.pallas import tpu as pltpu
from jax.experimental.pallas import tpu_sc as plsc

_F = 26
_D = 16
_B = 16384
_V1 = 100001
_NC = 2
_NS = 16
_L = 16
_NW = _NC * _NS
_BPW = _B // _NW
_CH = 128
_NCH = _BPW // _CH

_TBLK = 4096
_TGRID = -(-_V1 // _TBLK)
_ROWS_PAD = _TGRID * _TBLK
_HPB = _TBLK // 1024


def _make_relayout(nt):
    def body(*refs):
        ins, outs = refs[:nt], refs[nt:]
        for t in range(nt):
            x = ins[t][...]
            for h in range(_HPB):
                c = jnp.concatenate(
                    [x[:, (8 * h + j) * 128:(8 * h + j + 1) * 128]
                     for j in range(8)], axis=0)
                outs[t][h * 128:(h + 1) * 128, :] = c.T

    return pl.pallas_call(
        body,
        grid=(_TGRID,),
        in_specs=[pl.BlockSpec((_D, _TBLK), lambda j: (0, j))] * nt,
        out_specs=[pl.BlockSpec((_HPB * 128, 128), lambda j: (j, 0))] * nt,
        out_shape=[
            jax.ShapeDtypeStruct((_TGRID * _HPB * 128, 128), jnp.float32)
        ] * nt,
    )


_PF = 32


def _post_body(x_ref, o_ref):
    c = x_ref[...].T
    cr = c.reshape(16, 8, 128)
    for fr in range(8):
        for g in range(2):
            o_ref[fr, g, 0, :, :] = cr[2 * fr + g]


_postlayout = pl.pallas_call(
    _post_body,
    grid=(_B // 128, 4),
    in_specs=[pl.BlockSpec((128, 128), lambda bt, j: (j * (_B // 128) + bt, 0))],
    out_specs=pl.BlockSpec((8, 2, 1, 8, 128), lambda bt, j: (j, 0, bt, 0, 0)),
    out_shape=jax.ShapeDtypeStruct((_F, 2, _B // 128, 8, 128), jnp.float32),
)


def _make_emb_kernel(nf):
    def body(*refs):
        feats = refs[:nf]
        tables = refs[nf:2 * nf]
        out = refs[2 * nf]
        idx_v, rows_v, sem_idx, sem_g, sem_w = refs[2 * nf + 1:]

        wid = lax.axis_index("s") * _NC + lax.axis_index("c")
        base = wid * _BPW

        idx_copies = [
            pltpu.make_async_copy(
                feats[i].at[pl.ds(base + c * _CH, _CH)],
                idx_v.at[i, c], sem_idx)
            for i in range(nf)
            for c in range(_NCH)
        ]
        for cp in idx_copies:
            cp.start()
        for cp in idx_copies:
            cp.wait()

        for i in range(nf):
            def rbody(it, _, i=i):
                c = it >> 3
                k = it & 7
                v = idx_v[i, c, pl.ds(k * _L, _L)]
                idx_v[i, c, pl.ds(k * _L, _L)] = (
                    (v & -1024) + ((v & 127) << 3) + ((v >> 7) & 7))
                return 0

            lax.fori_loop(0, _NCH * (_CH // _L), rbody, 0)

        def gather_descs(i, p):
            return [
                pltpu.make_async_copy(
                    tables[i].at[idx_v.at[i, c]],
                    rows_v.at[p, pl.ds(c * _CH, _CH)],
                    sem_g.at[p])
                for c in range(_NCH)
            ]

        def write_desc(i, p):
            return pltpu.make_async_copy(
                rows_v.at[p],
                out.at[i // 8, pl.ds(base, _BPW), pl.ds((i % 8) * _D, _D)],
                sem_w.at[p])

        for i in range(nf):
            p = i % 2
            if i >= 2:
                write_desc(i - 2, p).wait()
            for g in gather_descs(i, p):
                g.start()
            if i >= 1:
                q = 1 - p
                for g in gather_descs(i - 1, q):
                    g.wait()
                write_desc(i - 1, q).start()

        p_last = (nf - 1) % 2
        for g in gather_descs(nf - 1, p_last):
            g.wait()
        write_desc(nf - 1, p_last).start()
        write_desc(nf - 2, 1 - p_last).wait()
        write_desc(nf - 1, p_last).wait()

    return functools.partial(
        pl.kernel,
        mesh=plsc.VectorSubcoreMesh(core_axis_name="c", subcore_axis_name="s"),
        out_type=jax.ShapeDtypeStruct((4, _B, _PF * _D // 4), jnp.float32),
        scratch_types=[
            pltpu.VMEM((nf, _NCH, _CH), jnp.int32),
            pltpu.VMEM((2, _BPW, _D), jnp.float32),
            pltpu.SemaphoreType.DMA,
            pltpu.SemaphoreType.DMA((2,)),
            pltpu.SemaphoreType.DMA((2,)),
        ],
        compiler_params=pltpu.CompilerParams(use_tc_tiling_on_sc=False),
    )(lambda *refs: body(*refs))


_relayout_full = _make_relayout(_F)
_emb_full = _make_emb_kernel(_F)


def kernel(feat_0, feat_1, feat_2, feat_3, feat_4, feat_5, feat_6, feat_7,
           feat_8, feat_9, feat_10, feat_11, feat_12, feat_13, feat_14,
           feat_15, feat_16, feat_17, feat_18, feat_19, feat_20, feat_21,
           feat_22, feat_23, feat_24, feat_25,
           W_0, W_1, W_2, W_3, W_4, W_5, W_6, W_7, W_8, W_9, W_10, W_11,
           W_12, W_13, W_14, W_15, W_16, W_17, W_18, W_19, W_20, W_21,
           W_22, W_23, W_24, W_25):
    feats = [
        feat_0, feat_1, feat_2, feat_3, feat_4, feat_5, feat_6, feat_7,
        feat_8, feat_9, feat_10, feat_11, feat_12, feat_13, feat_14,
        feat_15, feat_16, feat_17, feat_18, feat_19, feat_20, feat_21,
        feat_22, feat_23, feat_24, feat_25,
    ]
    tables = [
        W_0, W_1, W_2, W_3, W_4, W_5, W_6, W_7, W_8, W_9, W_10, W_11,
        W_12, W_13, W_14, W_15, W_16, W_17, W_18, W_19, W_20, W_21,
        W_22, W_23, W_24, W_25,
    ]
    d = _relayout_full(*[w.T for w in tables])
    dense = [t.reshape(_ROWS_PAD, _D) for t in d]
    out = _emb_full(*feats, *dense)
    out5 = _postlayout(out.reshape(4 * _B, 128))
    return out5.transpose(2, 4, 0, 1, 3).reshape(_B, _F, _D)


# speedup vs baseline: 1.9688x; 1.9683x over previous
import functools

import jax
import jax.numpy as jnp
from jax import lax
from jax.experimental import pallas as pl
from jax.experimental.pallas import tpu as pltpu
from jax.experimental.pallas import tpu_sc as plsc

_F = 26
_D = 16
_B = 16384
_V1 = 100001
_NC = 2
_NS = 16
_L = 16
_NW = _NC * _NS
_BPW = _B // _NW
_CH = 128
_NCH = _BPW // _CH

_TBLK = 4096
_TGRID = -(-_V1 // _TBLK)
_ROWS_PAD = _TGRID * _TBLK
_HPB = _TBLK // 1024


def _make_relayout(nt):
    def body(*refs):
        ins, outs = refs[:nt], refs[nt:]
        for t in range(nt):
            x = ins[t][...]
            for h in range(_HPB):
                c = jnp.concatenate(
                    [x[:, (8 * h + j) * 128:(8 * h + j + 1) * 128]
                     for j in range(8)], axis=0)
                outs[t][h * 128:(h + 1) * 128, :] = c.T

    return pl.pallas_call(
        body,
        grid=(_TGRID,),
        in_specs=[pl.BlockSpec((_D, _TBLK), lambda j: (0, j))] * nt,
        out_specs=[pl.BlockSpec((_HPB * 128, 128), lambda j: (j, 0))] * nt,
        out_shape=[
            jax.ShapeDtypeStruct((_TGRID * _HPB * 128, 128), jnp.float32)
        ] * nt,
    )


def _make_emb_kernel(nf):
    def body(*refs):
        feats = refs[:nf]
        tables = refs[nf:2 * nf]
        out = refs[2 * nf]
        idx_v, rows_v, sem_idx, sem_g, sem_w = refs[2 * nf + 1:]

        wid = lax.axis_index("s") * _NC + lax.axis_index("c")
        base = wid * _BPW

        idx_copies = [
            pltpu.make_async_copy(
                feats[i].at[pl.ds(base + c * _CH, _CH)],
                idx_v.at[i, c], sem_idx)
            for i in range(nf)
            for c in range(_NCH)
        ]
        for cp in idx_copies:
            cp.start()
        for cp in idx_copies:
            cp.wait()

        for i in range(nf):
            def rbody(it, _, i=i):
                c = it >> 3
                k = it & 7
                v = idx_v[i, c, pl.ds(k * _L, _L)]
                idx_v[i, c, pl.ds(k * _L, _L)] = (
                    (v & -1024) + ((v & 127) << 3) + ((v >> 7) & 7))
                return 0

            lax.fori_loop(0, _NCH * (_CH // _L), rbody, 0)

        def gather_descs(i, p):
            return [
                pltpu.make_async_copy(
                    tables[i].at[idx_v.at[i, c]],
                    rows_v.at[p, pl.ds(c * _CH, _CH)],
                    sem_g.at[p])
                for c in range(_NCH)
            ]

        def write_desc(i, p):
            return pltpu.make_async_copy(
                rows_v.at[p],
                out.at[pl.ds(base, _BPW), pl.ds(i * _D, _D)],
                sem_w.at[p])

        for i in range(nf):
            p = i % 2
            if i >= 2:
                write_desc(i - 2, p).wait()
            for g in gather_descs(i, p):
                g.start()
            if i >= 1:
                q = 1 - p
                for g in gather_descs(i - 1, q):
                    g.wait()
                write_desc(i - 1, q).start()

        p_last = (nf - 1) % 2
        for g in gather_descs(nf - 1, p_last):
            g.wait()
        write_desc(nf - 1, p_last).start()
        write_desc(nf - 2, 1 - p_last).wait()
        write_desc(nf - 1, p_last).wait()

    return functools.partial(
        pl.kernel,
        mesh=plsc.VectorSubcoreMesh(core_axis_name="c", subcore_axis_name="s"),
        out_type=jax.ShapeDtypeStruct((_B, nf * _D), jnp.float32),
        scratch_types=[
            pltpu.VMEM((nf, _NCH, _CH), jnp.int32),
            pltpu.VMEM((2, _BPW, _D), jnp.float32),
            pltpu.SemaphoreType.DMA,
            pltpu.SemaphoreType.DMA((2,)),
            pltpu.SemaphoreType.DMA((2,)),
        ],
        compiler_params=pltpu.CompilerParams(use_tc_tiling_on_sc=False),
    )(lambda *refs: body(*refs))


_relayout_full = _make_relayout(_F)
_emb_full = _make_emb_kernel(_F)


def kernel(feat_0, feat_1, feat_2, feat_3, feat_4, feat_5, feat_6, feat_7,
           feat_8, feat_9, feat_10, feat_11, feat_12, feat_13, feat_14,
           feat_15, feat_16, feat_17, feat_18, feat_19, feat_20, feat_21,
           feat_22, feat_23, feat_24, feat_25,
           W_0, W_1, W_2, W_3, W_4, W_5, W_6, W_7, W_8, W_9, W_10, W_11,
           W_12, W_13, W_14, W_15, W_16, W_17, W_18, W_19, W_20, W_21,
           W_22, W_23, W_24, W_25):
    feats = [
        feat_0, feat_1, feat_2, feat_3, feat_4, feat_5, feat_6, feat_7,
        feat_8, feat_9, feat_10, feat_11, feat_12, feat_13, feat_14,
        feat_15, feat_16, feat_17, feat_18, feat_19, feat_20, feat_21,
        feat_22, feat_23, feat_24, feat_25,
    ]
    tables = [
        W_0, W_1, W_2, W_3, W_4, W_5, W_6, W_7, W_8, W_9, W_10, W_11,
        W_12, W_13, W_14, W_15, W_16, W_17, W_18, W_19, W_20, W_21,
        W_22, W_23, W_24, W_25,
    ]
    d = _relayout_full(*[w.T for w in tables])
    dense = [t.reshape(_ROWS_PAD, _D) for t in d]
    out = _emb_full(*feats, *dense)
    return out.reshape(_B, _F, _D)
